# Initial kernel scaffold; baseline (speedup 1.0000x reference)
import jax
import jax.numpy as jnp
from jax.experimental import pallas as pl


def kernel(x, edge_index, y, batch, W1, b1, W2, b2, M1, mb1, M2, mb2):
    raise NotImplementedError("write your pallas kernel here")


# speedup vs baseline: 9.2150x; 9.2150x over previous
import functools

import jax
import jax.numpy as jnp
from jax import lax
from jax.experimental import pallas as pl
from jax.experimental.pallas import tpu as pltpu
from jax.experimental.pallas import tpu_sc as plsc

N = 10000
EDGES = 320000
GQ = 64
H = 128
NC, NS = 2, 16
NW = NC * NS
K = 128
GWIN = EDGES // K
NWIN = GWIN // NW
TAIL = GWIN - NWIN * NW
NP = 10240
RPT = NP // NS
BLK = 1000
GRID = N // BLK

@functools.lru_cache(maxsize=None)
def _sc_mesh():
    return plsc.VectorSubcoreMesh(
        core_axis_name="c", subcore_axis_name="s",
        num_cores=NC, num_subcores=NS)


@functools.lru_cache(maxsize=None)
def _make_spmm(width):
    @functools.partial(
        pl.kernel,
        out_type=jax.ShapeDtypeStruct((NC, NP, width), jnp.float32),
        mesh=_sc_mesh(),
        scratch_types=[
            pltpu.VMEM((K,), jnp.int32),
            pltpu.VMEM((K,), jnp.int32),
            pltpu.VMEM((K, width), jnp.float32),
            pltpu.VMEM_SHARED((NP, width), jnp.float32),
            pltpu.SemaphoreType.DMA,
        ],
    )
    def spmm(table, src, dst, zeros, out, src_v, dst_v, rows_v, acc, sem):
        c = lax.axis_index("c")
        s = lax.axis_index("s")
        wid = s * NC + c
        pltpu.sync_copy(zeros.at[pl.ds(s * RPT, RPT)],
                        acc.at[pl.ds(s * RPT, RPT)])
        plsc.subcore_barrier()

        def _win(g):
            off = g * K
            pltpu.sync_copy(src.at[pl.ds(off, K)], src_v)
            pltpu.sync_copy(dst.at[pl.ds(off, K)], dst_v)
            pltpu.async_copy(table.at[src_v], rows_v, sem).wait()
            pltpu.sync_copy(rows_v, acc.at[dst_v], add=True)

        @pl.loop(0, NWIN)
        def _full(w):
            _win(wid + w * NW)

        @pl.when(wid < TAIL)
        def _tail():
            _win(NWIN * NW + wid)

        plsc.subcore_barrier()
        pltpu.sync_copy(acc.at[pl.ds(s * RPT, RPT)],
                        out.at[c, pl.ds(s * RPT, RPT)])

    return spmm


@functools.lru_cache(maxsize=None)
def _make_deg_kernel():
    return functools.partial(
        pl.kernel,
        out_type=jax.ShapeDtypeStruct((NC, 1, NP), jnp.float32),
        mesh=_sc_mesh(),
        scratch_types=[
            pltpu.VMEM((K,), jnp.int32),
            pltpu.VMEM((K,), jnp.float32),
            pltpu.VMEM_SHARED((NP,), jnp.float32),
            pltpu.SemaphoreType.DMA,
        ],
    )(_deg_body)


def _deg_body(dst, zeros, out, dst_v, ones_v, acc, sem):
    c = lax.axis_index("c")
    s = lax.axis_index("s")
    wid = s * NC + c

    @pl.loop(0, K // 16)
    def _fill(i):
        ones_v[pl.ds(i * 16, 16)] = jnp.full((16,), 1.0, jnp.float32)

    pltpu.sync_copy(zeros.at[pl.ds(s * RPT, RPT)],
                    acc.at[pl.ds(s * RPT, RPT)])
    plsc.subcore_barrier()

    def _win(g):
        pltpu.sync_copy(dst.at[pl.ds(g * K, K)], dst_v)
        pltpu.sync_copy(ones_v, acc.at[dst_v], add=True)

    @pl.loop(0, NWIN)
    def _full(w):
        _win(wid + w * NW)

    @pl.when(wid < TAIL)
    def _tail():
        _win(NWIN * NW + wid)

    plsc.subcore_barrier()
    pltpu.sync_copy(acc.at[pl.ds(s * RPT, RPT)],
                    out.at[c, 0, pl.ds(s * RPT, RPT)])


def _row_spec(shape):
    nd = len(shape)
    if nd == 2:
        return pl.BlockSpec((BLK,) + shape[1:], lambda i: (i, 0))
    return pl.BlockSpec((shape[0], BLK) + shape[2:], lambda i: (0, i, 0))


def _full_spec(shape):
    return pl.BlockSpec(shape, lambda i: (0,) * len(shape))


def _pre_body(degp, batch2, y, x, W1_0, dinv_o, yb_o, hw1s_o):
    deg = jnp.sum(degp[...], axis=1, keepdims=True) + 1.0
    dinv = lax.rsqrt(deg)
    oh = (batch2[...] ==
          lax.broadcasted_iota(jnp.int32, (BLK, GQ), 1)).astype(jnp.float32)
    yb = jnp.dot(oh, y[...], preferred_element_type=jnp.float32,
                 precision=lax.Precision.HIGHEST)
    mask0 = (lax.broadcasted_iota(jnp.int32, (1, 4), 1) % 2
             == 0).astype(jnp.float32)
    zy = jnp.concatenate([x[...] * mask0, yb], axis=-1)
    hw1 = jnp.dot(zy, W1_0[...], preferred_element_type=jnp.float32)
    dinv_o[...] = dinv
    yb_o[...] = yb
    hw1s_o[...] = dinv * hw1


def _pre(degp, batch2, y, x, W1_0):
    return pl.pallas_call(
        _pre_body,
        grid=(GRID,),
        in_specs=[
            _row_spec((N, NC)),
            _row_spec((N, 1)),
            _full_spec((GQ, 2)),
            _row_spec((N, 4)),
            _full_spec((6, H)),
        ],
        out_specs=[_row_spec((N, 1)), _row_spec((N, 2)), _row_spec((N, H))],
        out_shape=[
            jax.ShapeDtypeStruct((N, 1), jnp.float32),
            jax.ShapeDtypeStruct((N, 2), jnp.float32),
            jax.ShapeDtypeStruct((N, H), jnp.float32),
        ],
    )(degp.reshape(NC, NP)[:, :N].T, batch2, y, x, W1_0)


def _mid_body(rawp, hw1s, dinv, b1, W2, hw2s_o):
    h1 = jnp.maximum(
        dinv[...] * (rawp[0] + rawp[1] + hw1s[...]) + b1[...], 0.0)
    hw2s_o[...] = dinv[...] * jnp.dot(
        h1, W2[...], preferred_element_type=jnp.float32)


def _mid(rawp, hw1s, dinv, b1, W2):
    return pl.pallas_call(
        _mid_body,
        grid=(GRID,),
        in_specs=[
            _row_spec((NC, N, H)),
            _row_spec((N, H)),
            _row_spec((N, 1)),
            _full_spec((1, H)),
            _full_spec((H, H)),
        ],
        out_specs=[_row_spec((N, H))],
        out_shape=[jax.ShapeDtypeStruct((N, H), jnp.float32)],
    )(rawp, hw1s, dinv, b1, W2)[0]


def _make_coupling(even):
    rem = 0 if even else 1

    def body(rawp, hw2s, z, ld, dinv, yb, b2, M1, mb1, M2, mb2, W1n,
             z_o, ld_o, hw1s_o):
        mask = (lax.broadcasted_iota(jnp.int32, (1, 4), 1) % 2
                == rem).astype(jnp.float32)
        h2 = jnp.maximum(
            dinv[...] * (rawp[0] + rawp[1] + hw2s[...]) + b2[...], 0.0)
        u = jnp.maximum(
            jnp.dot(h2, M1[...], preferred_element_type=jnp.float32)
            + mb1[...], 0.0)
        st = jnp.dot(u, M2[...],
                     preferred_element_type=jnp.float32) + mb2[...]
        inv = 1.0 - mask
        s = jnp.tanh(st[:, :4]) * inv
        t = st[:, 4:] * inv
        zc = z[...]
        zn = zc * mask + inv * (zc * jnp.exp(s) + t)
        z_o[...] = zn
        ld_o[...] = ld[...] + jnp.sum(s, axis=-1, keepdims=True)
        zy = jnp.concatenate([zn * inv, yb[...]], axis=-1)
        hw1s_o[...] = dinv[...] * jnp.dot(
            zy, W1n[...], preferred_element_type=jnp.float32)

    def call(rawp, hw2s, z, ld, dinv, yb, b2, M1, mb1, M2, mb2, W1n):
        return pl.pallas_call(
            body,
            grid=(GRID,),
            in_specs=[
                _row_spec((NC, N, H)),
                _row_spec((N, H)),
                _row_spec((N, 4)),
                _row_spec((N, 1)),
                _row_spec((N, 1)),
                _row_spec((N, 2)),
                _full_spec((1, H)),
                _full_spec((H, H)),
                _full_spec((1, H)),
                _full_spec((H, 8)),
                _full_spec((1, 8)),
                _full_spec((6, H)),
            ],
            out_specs=[_row_spec((N, 4)), _row_spec((N, 1)),
                       _row_spec((N, H))],
            out_shape=[
                jax.ShapeDtypeStruct((N, 4), jnp.float32),
                jax.ShapeDtypeStruct((N, 1), jnp.float32),
                jax.ShapeDtypeStruct((N, H), jnp.float32),
            ],
        )(rawp, hw2s, z, ld, dinv, yb, b2, M1, mb1, M2, mb2, W1n)

    return call


_coupling_even = _make_coupling(True)
_coupling_odd = _make_coupling(False)


def kernel(x, edge_index, y, batch, W1, b1, W2, b2, M1, mb1, M2, mb2):
    L = W1.shape[0]
    src = edge_index[0]
    dst = edge_index[1]
    zeros1 = jnp.zeros((NP,), jnp.float32)
    zerosH = jnp.zeros((NP, H), jnp.float32)
    batch2 = batch.reshape(N, 1)

    degp = _make_deg_kernel()(dst, zeros1)
    _spmmH = _make_spmm(H)
    dinv, yb, hw1s = _pre(degp, batch2, y, x, W1[0])

    b1r = b1.reshape(L, 1, H)
    b2r = b2.reshape(L, 1, H)
    mb1r = mb1.reshape(L, 1, H)
    mb2r = mb2.reshape(L, 1, 8)

    z = x
    ld = jnp.zeros((N, 1), jnp.float32)
    for i in range(L):
        raw1 = _spmmH(hw1s, src, dst, zerosH)
        hw2s = _mid(raw1, hw1s, dinv, b1r[i], W2[i])
        raw2 = _spmmH(hw2s, src, dst, zerosH)
        coup = _coupling_even if i % 2 == 0 else _coupling_odd
        z, ld, hw1s = coup(raw2, hw2s, z, ld, dinv, yb, b2r[i],
                           M1[i], mb1r[i], M2[i], mb2r[i], W1[(i + 1) % L])
    return z, ld.reshape(N)


# speedup vs baseline: 12.4480x; 1.3508x over previous
import functools

import jax
import jax.numpy as jnp
from jax import lax
from jax.experimental import pallas as pl
from jax.experimental.pallas import tpu as pltpu
from jax.experimental.pallas import tpu_sc as plsc

N = 10000
EDGES = 320000
GQ = 64
H = 128
NC, NS = 2, 16
NW = NC * NS
K = 128
NP = 10240
EP = 327680
NWT = EP // NW // K
RPT = NP // NS
BLK = 1000
GRID = N // BLK

@functools.lru_cache(maxsize=None)
def _sc_mesh():
    return plsc.VectorSubcoreMesh(
        core_axis_name="c", subcore_axis_name="s",
        num_cores=NC, num_subcores=NS)


@functools.lru_cache(maxsize=None)
def _make_spmm(width):
    @functools.partial(
        pl.kernel,
        out_type=jax.ShapeDtypeStruct((NC, NP, width), jnp.float32),
        mesh=_sc_mesh(),
        scratch_types=[
            pltpu.VMEM((2, K), jnp.int32),
            pltpu.VMEM((2, K), jnp.int32),
            pltpu.VMEM((2, K, width), jnp.float32),
            pltpu.VMEM_SHARED((NP, width), jnp.float32),
            pltpu.SemaphoreType.DMA,
            pltpu.SemaphoreType.DMA,
            pltpu.SemaphoreType.DMA,
            pltpu.SemaphoreType.DMA,
        ],
    )
    def spmm(table, src1, dst1, zeros, out, src_v, dst_v, rows_v, acc,
             gsem0, gsem1, isem0, isem1):
        c = lax.axis_index("c")
        s = lax.axis_index("s")
        wid = s * NC + c
        base = wid * NWT * K
        gsems = (gsem0, gsem1)
        isems = (isem0, isem1)

        def idx_start(wi, b):
            off = base + wi * K
            pltpu.async_copy(src1.at[pl.ds(off, K)], src_v.at[b], isems[b])
            pltpu.async_copy(dst1.at[pl.ds(off, K)], dst_v.at[b], isems[b])

        def idx_wait(wi, b):
            off = base + wi * K
            pltpu.make_async_copy(
                src1.at[pl.ds(off, K)], src_v.at[b], isems[b]).wait()
            pltpu.make_async_copy(
                dst1.at[pl.ds(off, K)], dst_v.at[b], isems[b]).wait()

        def gather_start(b):
            pltpu.async_copy(table.at[src_v.at[b]], rows_v.at[b], gsems[b])

        pltpu.sync_copy(zeros.at[pl.ds(s * RPT, RPT)],
                        acc.at[pl.ds(s * RPT, RPT)])
        idx_start(0, 0)
        idx_start(1, 1)
        plsc.subcore_barrier()
        idx_wait(0, 0)
        gather_start(0)

        @pl.loop(0, NWT, step=2)
        def _pair(w):
            for b in range(2):
                wi = w + b
                pltpu.make_async_copy(
                    table.at[src_v.at[b]], rows_v.at[b], gsems[b]).wait()
                pltpu.sync_copy(rows_v.at[b], acc.at[dst_v.at[b]],
                                add=True)

                @pl.when(wi + 2 < NWT)
                def _prefetch():
                    idx_start(wi + 2, b)

                @pl.when(wi + 1 < NWT)
                def _next_gather():
                    idx_wait(wi + 1, 1 - b)
                    gather_start(1 - b)

        plsc.subcore_barrier()
        pltpu.sync_copy(acc.at[pl.ds(s * RPT, RPT)],
                        out.at[c, pl.ds(s * RPT, RPT)])

    return spmm


@functools.lru_cache(maxsize=None)
def _make_deg_kernel():
    return functools.partial(
        pl.kernel,
        out_type=jax.ShapeDtypeStruct((NC, 1, NP), jnp.float32),
        mesh=_sc_mesh(),
        scratch_types=[
            pltpu.VMEM((NWT, K), jnp.int32),
            pltpu.VMEM((K,), jnp.float32),
            pltpu.VMEM_SHARED((NP,), jnp.float32),
            pltpu.SemaphoreType.DMA,
        ],
    )(_deg_body)


def _deg_body(dst2, zeros, out, dst_s, ones_v, acc, sem):
    c = lax.axis_index("c")
    s = lax.axis_index("s")
    wid = s * NC + c

    @pl.loop(0, K // 16)
    def _fill(i):
        ones_v[pl.ds(i * 16, 16)] = jnp.full((16,), 1.0, jnp.float32)

    pltpu.sync_copy(zeros.at[pl.ds(s * RPT, RPT)],
                    acc.at[pl.ds(s * RPT, RPT)])
    pltpu.sync_copy(dst2.at[pl.ds(wid * NWT, NWT)], dst_s)
    plsc.subcore_barrier()

    @pl.loop(0, NWT)
    def _win(w):
        pltpu.sync_copy(ones_v, acc.at[dst_s.at[w]], add=True)

    plsc.subcore_barrier()
    pltpu.sync_copy(acc.at[pl.ds(s * RPT, RPT)],
                    out.at[c, 0, pl.ds(s * RPT, RPT)])


def _row_spec(shape):
    nd = len(shape)
    if nd == 2:
        return pl.BlockSpec((BLK,) + shape[1:], lambda i: (i, 0))
    return pl.BlockSpec((shape[0], BLK) + shape[2:], lambda i: (0, i, 0))


def _full_spec(shape):
    return pl.BlockSpec(shape, lambda i: (0,) * len(shape))


def _pre_body(degp, batch2, y, x, W1_0, dinv_o, yb_o, hw1s_o):
    deg = jnp.sum(degp[...], axis=1, keepdims=True) + 1.0
    dinv = lax.rsqrt(deg)
    oh = (batch2[...] ==
          lax.broadcasted_iota(jnp.int32, (BLK, GQ), 1)).astype(jnp.float32)
    yb = jnp.dot(oh, y[...], preferred_element_type=jnp.float32,
                 precision=lax.Precision.HIGHEST)
    mask0 = (lax.broadcasted_iota(jnp.int32, (1, 4), 1) % 2
             == 0).astype(jnp.float32)
    zy = jnp.concatenate([x[...] * mask0, yb], axis=-1)
    hw1 = jnp.dot(zy, W1_0[...], preferred_element_type=jnp.float32)
    dinv_o[...] = dinv
    yb_o[...] = yb
    hw1s_o[...] = dinv * hw1


def _pre(degp, batch2, y, x, W1_0):
    return pl.pallas_call(
        _pre_body,
        grid=(GRID,),
        in_specs=[
            _row_spec((N, NC)),
            _row_spec((N, 1)),
            _full_spec((GQ, 2)),
            _row_spec((N, 4)),
            _full_spec((6, H)),
        ],
        out_specs=[_row_spec((N, 1)), _row_spec((N, 2)), _row_spec((N, H))],
        out_shape=[
            jax.ShapeDtypeStruct((N, 1), jnp.float32),
            jax.ShapeDtypeStruct((N, 2), jnp.float32),
            jax.ShapeDtypeStruct((N, H), jnp.float32),
        ],
    )(degp.reshape(NC, NP)[:, :N].T, batch2, y, x, W1_0)


def _mid_body(rawp, hw1s, dinv, b1, W2, hw2s_o):
    h1 = jnp.maximum(
        dinv[...] * (rawp[0] + rawp[1] + hw1s[...]) + b1[...], 0.0)
    hw2s_o[...] = dinv[...] * jnp.dot(
        h1, W2[...], preferred_element_type=jnp.float32)


def _mid(rawp, hw1s, dinv, b1, W2):
    return pl.pallas_call(
        _mid_body,
        grid=(GRID,),
        in_specs=[
            _row_spec((NC, N, H)),
            _row_spec((N, H)),
            _row_spec((N, 1)),
            _full_spec((1, H)),
            _full_spec((H, H)),
        ],
        out_specs=[_row_spec((N, H))],
        out_shape=[jax.ShapeDtypeStruct((N, H), jnp.float32)],
    )(rawp, hw1s, dinv, b1, W2)[0]


def _make_coupling(even):
    rem = 0 if even else 1

    def body(rawp, hw2s, z, ld, dinv, yb, b2, M1, mb1, M2, mb2, W1n,
             z_o, ld_o, hw1s_o):
        mask = (lax.broadcasted_iota(jnp.int32, (1, 4), 1) % 2
                == rem).astype(jnp.float32)
        h2 = jnp.maximum(
            dinv[...] * (rawp[0] + rawp[1] + hw2s[...]) + b2[...], 0.0)
        u = jnp.maximum(
            jnp.dot(h2, M1[...], preferred_element_type=jnp.float32)
            + mb1[...], 0.0)
        st = jnp.dot(u, M2[...],
                     preferred_element_type=jnp.float32) + mb2[...]
        inv = 1.0 - mask
        s = jnp.tanh(st[:, :4]) * inv
        t = st[:, 4:] * inv
        zc = z[...]
        zn = zc * mask + inv * (zc * jnp.exp(s) + t)
        z_o[...] = zn
        ld_o[...] = ld[...] + jnp.sum(s, axis=-1, keepdims=True)
        zy = jnp.concatenate([zn * inv, yb[...]], axis=-1)
        hw1s_o[...] = dinv[...] * jnp.dot(
            zy, W1n[...], preferred_element_type=jnp.float32)

    def call(rawp, hw2s, z, ld, dinv, yb, b2, M1, mb1, M2, mb2, W1n):
        return pl.pallas_call(
            body,
            grid=(GRID,),
            in_specs=[
                _row_spec((NC, N, H)),
                _row_spec((N, H)),
                _row_spec((N, 4)),
                _row_spec((N, 1)),
                _row_spec((N, 1)),
                _row_spec((N, 2)),
                _full_spec((1, H)),
                _full_spec((H, H)),
                _full_spec((1, H)),
                _full_spec((H, 8)),
                _full_spec((1, 8)),
                _full_spec((6, H)),
            ],
            out_specs=[_row_spec((N, 4)), _row_spec((N, 1)),
                       _row_spec((N, H))],
            out_shape=[
                jax.ShapeDtypeStruct((N, 4), jnp.float32),
                jax.ShapeDtypeStruct((N, 1), jnp.float32),
                jax.ShapeDtypeStruct((N, H), jnp.float32),
            ],
        )(rawp, hw2s, z, ld, dinv, yb, b2, M1, mb1, M2, mb2, W1n)

    return call


_coupling_even = _make_coupling(True)
_coupling_odd = _make_coupling(False)


def kernel(x, edge_index, y, batch, W1, b1, W2, b2, M1, mb1, M2, mb2):
    L = W1.shape[0]
    pad = jnp.arange(EP - EDGES, dtype=jnp.int32)
    src1 = jnp.concatenate([edge_index[0], pad % N])
    dst1 = jnp.concatenate([edge_index[1], N + pad % (NP - N)])
    dst2 = dst1.reshape(EP // K, K)
    zeros1 = jnp.zeros((NP,), jnp.float32)
    zerosH = jnp.zeros((NP, H), jnp.float32)
    batch2 = batch.reshape(N, 1)

    degp = _make_deg_kernel()(dst2, zeros1)
    _spmmH = _make_spmm(H)
    dinv, yb, hw1s = _pre(degp, batch2, y, x, W1[0])

    b1r = b1.reshape(L, 1, H)
    b2r = b2.reshape(L, 1, H)
    mb1r = mb1.reshape(L, 1, H)
    mb2r = mb2.reshape(L, 1, 8)

    z = x
    ld = jnp.zeros((N, 1), jnp.float32)
    for i in range(L):
        raw1 = _spmmH(hw1s, src1, dst1, zerosH)
        hw2s = _mid(raw1, hw1s, dinv, b1r[i], W2[i])
        raw2 = _spmmH(hw2s, src1, dst1, zerosH)
        coup = _coupling_even if i % 2 == 0 else _coupling_odd
        z, ld, hw1s = coup(raw2, hw2s, z, ld, dinv, yb, b2r[i],
                           M1[i], mb1r[i], M2[i], mb2r[i], W1[(i + 1) % L])
    return z, ld.reshape(N)


# speedup vs baseline: 16.2331x; 1.3041x over previous
import functools

import jax
import jax.numpy as jnp
from jax import lax
from jax.experimental import pallas as pl
from jax.experimental.pallas import tpu as pltpu
from jax.experimental.pallas import tpu_sc as plsc

N = 10000
EDGES = 320000
GQ = 64
H = 128
NC, NS = 2, 16
NW = NC * NS
K = 128
NP = 10240
EP = 327680
NWT = EP // NW // K
RPT = NP // NS
BLK = 1000
GRID = N // BLK

@functools.lru_cache(maxsize=None)
def _sc_mesh():
    return plsc.VectorSubcoreMesh(
        core_axis_name="c", subcore_axis_name="s",
        num_cores=NC, num_subcores=NS)


@functools.lru_cache(maxsize=None)
def _make_spmm(width):
    @functools.partial(
        pl.kernel,
        out_type=jax.ShapeDtypeStruct((NC, NP, width), jnp.float32),
        mesh=_sc_mesh(),
        scratch_types=[
            pltpu.VMEM((2, K), jnp.int32),
            pltpu.VMEM((2, K), jnp.int32),
            pltpu.VMEM((2, K, width), jnp.float32),
            pltpu.VMEM_SHARED((NP, width), jnp.float32),
            pltpu.SemaphoreType.DMA,
            pltpu.SemaphoreType.DMA,
            pltpu.SemaphoreType.DMA,
            pltpu.SemaphoreType.DMA,
        ],
    )
    def spmm(table, src1, dst1, zeros, out, src_v, dst_v, rows_v, acc,
             gsem0, gsem1, isem0, isem1):
        c = lax.axis_index("c")
        s = lax.axis_index("s")
        wid = s * NC + c
        base = wid * NWT * K
        gsems = (gsem0, gsem1)
        isems = (isem0, isem1)

        def idx_start(wi, b):
            off = base + wi * K
            pltpu.async_copy(src1.at[pl.ds(off, K)], src_v.at[b], isems[b])
            pltpu.async_copy(dst1.at[pl.ds(off, K)], dst_v.at[b], isems[b])

        def idx_wait(wi, b):
            off = base + wi * K
            pltpu.make_async_copy(
                src1.at[pl.ds(off, K)], src_v.at[b], isems[b]).wait()
            pltpu.make_async_copy(
                dst1.at[pl.ds(off, K)], dst_v.at[b], isems[b]).wait()

        def gather_start(b):
            pltpu.async_copy(table.at[src_v.at[b]], rows_v.at[b], gsems[b])

        pltpu.sync_copy(zeros.at[pl.ds(s * RPT, RPT)],
                        acc.at[pl.ds(s * RPT, RPT)])
        idx_start(0, 0)
        idx_start(1, 1)
        plsc.subcore_barrier()
        idx_wait(0, 0)
        gather_start(0)

        @pl.loop(0, NWT, step=2)
        def _pair(w):
            for b in range(2):
                wi = w + b
                pltpu.make_async_copy(
                    table.at[src_v.at[b]], rows_v.at[b], gsems[b]).wait()

                @pl.when(wi + 1 < NWT)
                def _next_gather():
                    idx_wait(wi + 1, 1 - b)
                    gather_start(1 - b)

                pltpu.sync_copy(rows_v.at[b], acc.at[dst_v.at[b]],
                                add=True)

                @pl.when(wi + 2 < NWT)
                def _prefetch():
                    idx_start(wi + 2, b)

        plsc.subcore_barrier()
        pltpu.sync_copy(acc.at[pl.ds(s * RPT, RPT)],
                        out.at[c, pl.ds(s * RPT, RPT)])

    return spmm


@functools.lru_cache(maxsize=None)
def _make_deg_kernel():
    return functools.partial(
        pl.kernel,
        out_type=jax.ShapeDtypeStruct((NC, 1, NP), jnp.float32),
        mesh=_sc_mesh(),
        scratch_types=[
            pltpu.VMEM((NWT, K), jnp.int32),
            pltpu.VMEM((K,), jnp.float32),
            pltpu.VMEM_SHARED((NP,), jnp.float32),
            pltpu.SemaphoreType.DMA,
        ],
    )(_deg_body)


def _deg_body(dst2, zeros, out, dst_s, ones_v, acc, sem):
    c = lax.axis_index("c")
    s = lax.axis_index("s")
    wid = s * NC + c

    @pl.loop(0, K // 16)
    def _fill(i):
        ones_v[pl.ds(i * 16, 16)] = jnp.full((16,), 1.0, jnp.float32)

    pltpu.sync_copy(zeros.at[pl.ds(s * RPT, RPT)],
                    acc.at[pl.ds(s * RPT, RPT)])
    pltpu.sync_copy(dst2.at[pl.ds(wid * NWT, NWT)], dst_s)
    plsc.subcore_barrier()

    @pl.loop(0, NWT)
    def _win(w):
        pltpu.sync_copy(ones_v, acc.at[dst_s.at[w]], add=True)

    plsc.subcore_barrier()
    pltpu.sync_copy(acc.at[pl.ds(s * RPT, RPT)],
                    out.at[c, 0, pl.ds(s * RPT, RPT)])


def _row_spec(shape):
    nd = len(shape)
    if nd == 2:
        return pl.BlockSpec((BLK,) + shape[1:], lambda i: (i, 0))
    return pl.BlockSpec((shape[0], BLK) + shape[2:], lambda i: (0, i, 0))


def _full_spec(shape):
    return pl.BlockSpec(shape, lambda i: (0,) * len(shape))


def _pre_body(degp, batch2, y, x, W1_0, dinv_o, yb_o, hw1s_o):
    deg = jnp.sum(degp[...], axis=1, keepdims=True) + 1.0
    dinv = lax.rsqrt(deg)
    oh = (batch2[...] ==
          lax.broadcasted_iota(jnp.int32, (BLK, GQ), 1)).astype(jnp.float32)
    yb = jnp.dot(oh, y[...], preferred_element_type=jnp.float32,
                 precision=lax.Precision.HIGHEST)
    mask0 = (lax.broadcasted_iota(jnp.int32, (1, 4), 1) % 2
             == 0).astype(jnp.float32)
    zy = jnp.concatenate([x[...] * mask0, yb], axis=-1)
    hw1 = jnp.dot(zy, W1_0[...], preferred_element_type=jnp.float32)
    dinv_o[...] = dinv
    yb_o[...] = yb
    hw1s_o[...] = dinv * hw1


def _pre(degp, batch2, y, x, W1_0):
    return pl.pallas_call(
        _pre_body,
        grid=(GRID,),
        in_specs=[
            _row_spec((N, NC)),
            _row_spec((N, 1)),
            _full_spec((GQ, 2)),
            _row_spec((N, 4)),
            _full_spec((6, H)),
        ],
        out_specs=[_row_spec((N, 1)), _row_spec((N, 2)), _row_spec((N, H))],
        out_shape=[
            jax.ShapeDtypeStruct((N, 1), jnp.float32),
            jax.ShapeDtypeStruct((N, 2), jnp.float32),
            jax.ShapeDtypeStruct((N, H), jnp.float32),
        ],
    )(degp.reshape(NC, NP)[:, :N].T, batch2, y, x, W1_0)


def _mid_body(rawp, hw1s, dinv, b1, W2, hw2s_o):
    h1 = jnp.maximum(
        dinv[...] * (rawp[0] + rawp[1] + hw1s[...]) + b1[...], 0.0)
    hw2s_o[...] = dinv[...] * jnp.dot(
        h1, W2[...], preferred_element_type=jnp.float32)


def _mid(rawp, hw1s, dinv, b1, W2):
    return pl.pallas_call(
        _mid_body,
        grid=(GRID,),
        in_specs=[
            _row_spec((NC, N, H)),
            _row_spec((N, H)),
            _row_spec((N, 1)),
            _full_spec((1, H)),
            _full_spec((H, H)),
        ],
        out_specs=[_row_spec((N, H))],
        out_shape=[jax.ShapeDtypeStruct((N, H), jnp.float32)],
    )(rawp, hw1s, dinv, b1, W2)[0]


def _make_coupling(even):
    rem = 0 if even else 1

    def body(rawp, hw2s, z, ld, dinv, yb, b2, M1, mb1, M2, mb2, W1n,
             z_o, ld_o, hw1s_o):
        mask = (lax.broadcasted_iota(jnp.int32, (1, 4), 1) % 2
                == rem).astype(jnp.float32)
        h2 = jnp.maximum(
            dinv[...] * (rawp[0] + rawp[1] + hw2s[...]) + b2[...], 0.0)
        u = jnp.maximum(
            jnp.dot(h2, M1[...], preferred_element_type=jnp.float32)
            + mb1[...], 0.0)
        st = jnp.dot(u, M2[...],
                     preferred_element_type=jnp.float32) + mb2[...]
        inv = 1.0 - mask
        s = jnp.tanh(st[:, :4]) * inv
        t = st[:, 4:] * inv
        zc = z[...]
        zn = zc * mask + inv * (zc * jnp.exp(s) + t)
        z_o[...] = zn
        ld_o[...] = ld[...] + jnp.sum(s, axis=-1, keepdims=True)
        zy = jnp.concatenate([zn * inv, yb[...]], axis=-1)
        hw1s_o[...] = dinv[...] * jnp.dot(
            zy, W1n[...], preferred_element_type=jnp.float32)

    def call(rawp, hw2s, z, ld, dinv, yb, b2, M1, mb1, M2, mb2, W1n):
        return pl.pallas_call(
            body,
            grid=(GRID,),
            in_specs=[
                _row_spec((NC, N, H)),
                _row_spec((N, H)),
                _row_spec((N, 4)),
                _row_spec((N, 1)),
                _row_spec((N, 1)),
                _row_spec((N, 2)),
                _full_spec((1, H)),
                _full_spec((H, H)),
                _full_spec((1, H)),
                _full_spec((H, 8)),
                _full_spec((1, 8)),
                _full_spec((6, H)),
            ],
            out_specs=[_row_spec((N, 4)), _row_spec((N, 1)),
                       _row_spec((N, H))],
            out_shape=[
                jax.ShapeDtypeStruct((N, 4), jnp.float32),
                jax.ShapeDtypeStruct((N, 1), jnp.float32),
                jax.ShapeDtypeStruct((N, H), jnp.float32),
            ],
        )(rawp, hw2s, z, ld, dinv, yb, b2, M1, mb1, M2, mb2, W1n)

    return call


_coupling_even = _make_coupling(True)
_coupling_odd = _make_coupling(False)


def kernel(x, edge_index, y, batch, W1, b1, W2, b2, M1, mb1, M2, mb2):
    L = W1.shape[0]
    pad = jnp.arange(EP - EDGES, dtype=jnp.int32)
    src1 = jnp.concatenate([edge_index[0], pad % N])
    dst1 = jnp.concatenate([edge_index[1], N + pad % (NP - N)])
    dst2 = dst1.reshape(EP // K, K)
    zeros1 = jnp.zeros((NP,), jnp.float32)
    zerosH = jnp.zeros((NP, H), jnp.float32)
    batch2 = batch.reshape(N, 1)

    degp = _make_deg_kernel()(dst2, zeros1)
    _spmmH = _make_spmm(H)
    dinv, yb, hw1s = _pre(degp, batch2, y, x, W1[0])

    b1r = b1.reshape(L, 1, H)
    b2r = b2.reshape(L, 1, H)
    mb1r = mb1.reshape(L, 1, H)
    mb2r = mb2.reshape(L, 1, 8)

    z = x
    ld = jnp.zeros((N, 1), jnp.float32)
    for i in range(L):
        raw1 = _spmmH(hw1s, src1, dst1, zerosH)
        hw2s = _mid(raw1, hw1s, dinv, b1r[i], W2[i])
        raw2 = _spmmH(hw2s, src1, dst1, zerosH)
        coup = _coupling_even if i % 2 == 0 else _coupling_odd
        z, ld, hw1s = coup(raw2, hw2s, z, ld, dinv, yb, b2r[i],
                           M1[i], mb1r[i], M2[i], mb2r[i], W1[(i + 1) % L])
    return z, ld.reshape(N)


# speedup vs baseline: 17.4556x; 1.0753x over previous
import functools

import jax
import jax.numpy as jnp
from jax import lax
from jax.experimental import pallas as pl
from jax.experimental.pallas import tpu as pltpu
from jax.experimental.pallas import tpu_sc as plsc

N = 10000
EDGES = 320000
GQ = 64
H = 128
NC, NS = 2, 16
NW = NC * NS
K = 128
NP = 10112
EP = 327680
NWT = EP // NW // K
RPT = NP // NS
BLK = 1000
GRID = N // BLK

@functools.lru_cache(maxsize=None)
def _sc_mesh():
    return plsc.VectorSubcoreMesh(
        core_axis_name="c", subcore_axis_name="s",
        num_cores=NC, num_subcores=NS)


@functools.lru_cache(maxsize=None)
def _make_spmm(width):
    @functools.partial(
        pl.kernel,
        out_type=jax.ShapeDtypeStruct((NC, NP, width), jnp.float32),
        mesh=_sc_mesh(),
        scratch_types=[
            pltpu.VMEM((3, K), jnp.int32),
            pltpu.VMEM((3, K), jnp.int32),
            pltpu.VMEM((3, K, width), jnp.float32),
            pltpu.VMEM_SHARED((NP, width), jnp.float32),
            pltpu.SemaphoreType.DMA,
            pltpu.SemaphoreType.DMA,
            pltpu.SemaphoreType.DMA,
            pltpu.SemaphoreType.DMA,
            pltpu.SemaphoreType.DMA,
            pltpu.SemaphoreType.DMA,
        ],
    )
    def spmm(table, src1, dst1, zeros, out, src_v, dst_v, rows_v, acc,
             gsem0, gsem1, gsem2, isem0, isem1, isem2):
        c = lax.axis_index("c")
        s = lax.axis_index("s")
        wid = s * NC + c
        base = wid * NWT * K
        gsems = (gsem0, gsem1, gsem2)
        isems = (isem0, isem1, isem2)

        def idx_start(wi, b):
            off = base + wi * K
            pltpu.async_copy(src1.at[pl.ds(off, K)], src_v.at[b], isems[b])
            pltpu.async_copy(dst1.at[pl.ds(off, K)], dst_v.at[b], isems[b])

        def idx_wait(wi, b):
            off = base + wi * K
            pltpu.make_async_copy(
                src1.at[pl.ds(off, K)], src_v.at[b], isems[b]).wait()
            pltpu.make_async_copy(
                dst1.at[pl.ds(off, K)], dst_v.at[b], isems[b]).wait()

        def gather_start(b):
            pltpu.async_copy(table.at[src_v.at[b]], rows_v.at[b], gsems[b])

        def gather_wait(b):
            pltpu.make_async_copy(
                table.at[src_v.at[b]], rows_v.at[b], gsems[b]).wait()

        pltpu.sync_copy(zeros.at[pl.ds(s * RPT, RPT)],
                        acc.at[pl.ds(s * RPT, RPT)])
        for b in range(3):
            idx_start(b, b)
        plsc.subcore_barrier()
        idx_wait(0, 0)
        gather_start(0)
        idx_wait(1, 1)
        gather_start(1)

        @pl.loop(0, NWT, step=3)
        def _trip(w):
            for b in range(3):
                wi = w + b
                nb = (b + 2) % 3

                @pl.when(wi + 2 < NWT)
                def _issue():
                    idx_wait(wi + 2, nb)
                    gather_start(nb)

                @pl.when(wi < NWT)
                def _work():
                    gather_wait(b)
                    pltpu.sync_copy(rows_v.at[b], acc.at[dst_v.at[b]],
                                    add=True)

                @pl.when(wi + 3 < NWT)
                def _prefetch():
                    idx_start(wi + 3, b)

        plsc.subcore_barrier()
        pltpu.sync_copy(acc.at[pl.ds(s * RPT, RPT)],
                        out.at[c, pl.ds(s * RPT, RPT)])

    return spmm


@functools.lru_cache(maxsize=None)
def _make_deg_kernel():
    return functools.partial(
        pl.kernel,
        out_type=jax.ShapeDtypeStruct((NC, 1, NP), jnp.float32),
        mesh=_sc_mesh(),
        scratch_types=[
            pltpu.VMEM((NWT, K), jnp.int32),
            pltpu.VMEM((K,), jnp.float32),
            pltpu.VMEM_SHARED((NP,), jnp.float32),
            pltpu.SemaphoreType.DMA,
        ],
    )(_deg_body)


def _deg_body(dst2, zeros, out, dst_s, ones_v, acc, sem):
    c = lax.axis_index("c")
    s = lax.axis_index("s")
    wid = s * NC + c

    @pl.loop(0, K // 16)
    def _fill(i):
        ones_v[pl.ds(i * 16, 16)] = jnp.full((16,), 1.0, jnp.float32)

    @pl.when(s < 15)
    def _init_a():
        pltpu.sync_copy(zeros.at[pl.ds(s * 640, 640)],
                        acc.at[pl.ds(s * 640, 640)])

    @pl.when(s == 15)
    def _init_b():
        pltpu.sync_copy(zeros.at[pl.ds(9600, NP - 9600)],
                        acc.at[pl.ds(9600, NP - 9600)])

    pltpu.sync_copy(dst2.at[pl.ds(wid * NWT, NWT)], dst_s)
    plsc.subcore_barrier()

    @pl.loop(0, NWT)
    def _win(w):
        pltpu.sync_copy(ones_v, acc.at[dst_s.at[w]], add=True)

    plsc.subcore_barrier()

    @pl.when(s < 15)
    def _flush_a():
        pltpu.sync_copy(acc.at[pl.ds(s * 640, 640)],
                        out.at[c, 0, pl.ds(s * 640, 640)])

    @pl.when(s == 15)
    def _flush_b():
        pltpu.sync_copy(acc.at[pl.ds(9600, NP - 9600)],
                        out.at[c, 0, pl.ds(9600, NP - 9600)])


def _row_spec(shape):
    nd = len(shape)
    if nd == 2:
        return pl.BlockSpec((BLK,) + shape[1:], lambda i: (i, 0))
    return pl.BlockSpec((shape[0], BLK) + shape[2:], lambda i: (0, i, 0))


def _full_spec(shape):
    return pl.BlockSpec(shape, lambda i: (0,) * len(shape))


def _pre_body(degp, batch2, y, x, W1_0, dinv_o, yb_o, hw1s_o):
    deg = jnp.sum(degp[...], axis=1, keepdims=True) + 1.0
    dinv = lax.rsqrt(deg)
    oh = (batch2[...] ==
          lax.broadcasted_iota(jnp.int32, (BLK, GQ), 1)).astype(jnp.float32)
    yb = jnp.dot(oh, y[...], preferred_element_type=jnp.float32,
                 precision=lax.Precision.HIGHEST)
    mask0 = (lax.broadcasted_iota(jnp.int32, (1, 4), 1) % 2
             == 0).astype(jnp.float32)
    zy = jnp.concatenate([x[...] * mask0, yb], axis=-1)
    hw1 = jnp.dot(zy, W1_0[...], preferred_element_type=jnp.float32)
    dinv_o[...] = dinv
    yb_o[...] = yb
    hw1s_o[...] = dinv * hw1


def _pre(degp, batch2, y, x, W1_0):
    return pl.pallas_call(
        _pre_body,
        grid=(GRID,),
        in_specs=[
            _row_spec((N, NC)),
            _row_spec((N, 1)),
            _full_spec((GQ, 2)),
            _row_spec((N, 4)),
            _full_spec((6, H)),
        ],
        out_specs=[_row_spec((N, 1)), _row_spec((N, 2)), _row_spec((N, H))],
        out_shape=[
            jax.ShapeDtypeStruct((N, 1), jnp.float32),
            jax.ShapeDtypeStruct((N, 2), jnp.float32),
            jax.ShapeDtypeStruct((N, H), jnp.float32),
        ],
    )(degp.reshape(NC, NP)[:, :N].T, batch2, y, x, W1_0)


def _mid_body(rawp, hw1s, dinv, b1, W2, hw2s_o):
    h1 = jnp.maximum(
        dinv[...] * (rawp[0] + rawp[1] + hw1s[...]) + b1[...], 0.0)
    hw2s_o[...] = dinv[...] * jnp.dot(
        h1, W2[...], preferred_element_type=jnp.float32)


def _mid(rawp, hw1s, dinv, b1, W2):
    return pl.pallas_call(
        _mid_body,
        grid=(GRID,),
        in_specs=[
            _row_spec((NC, N, H)),
            _row_spec((N, H)),
            _row_spec((N, 1)),
            _full_spec((1, H)),
            _full_spec((H, H)),
        ],
        out_specs=[_row_spec((N, H))],
        out_shape=[jax.ShapeDtypeStruct((N, H), jnp.float32)],
    )(rawp, hw1s, dinv, b1, W2)[0]


def _make_coupling(even):
    rem = 0 if even else 1

    def body(rawp, hw2s, z, ld, dinv, yb, b2, M1, mb1, M2, mb2, W1n,
             z_o, ld_o, hw1s_o):
        mask = (lax.broadcasted_iota(jnp.int32, (1, 4), 1) % 2
                == rem).astype(jnp.float32)
        h2 = jnp.maximum(
            dinv[...] * (rawp[0] + rawp[1] + hw2s[...]) + b2[...], 0.0)
        u = jnp.maximum(
            jnp.dot(h2, M1[...], preferred_element_type=jnp.float32)
            + mb1[...], 0.0)
        st = jnp.dot(u, M2[...],
                     preferred_element_type=jnp.float32) + mb2[...]
        inv = 1.0 - mask
        s = jnp.tanh(st[:, :4]) * inv
        t = st[:, 4:] * inv
        zc = z[...]
        zn = zc * mask + inv * (zc * jnp.exp(s) + t)
        z_o[...] = zn
        ld_o[...] = ld[...] + jnp.sum(s, axis=-1, keepdims=True)
        zy = jnp.concatenate([zn * inv, yb[...]], axis=-1)
        hw1s_o[...] = dinv[...] * jnp.dot(
            zy, W1n[...], preferred_element_type=jnp.float32)

    def call(rawp, hw2s, z, ld, dinv, yb, b2, M1, mb1, M2, mb2, W1n):
        return pl.pallas_call(
            body,
            grid=(GRID,),
            in_specs=[
                _row_spec((NC, N, H)),
                _row_spec((N, H)),
                _row_spec((N, 4)),
                _row_spec((N, 1)),
                _row_spec((N, 1)),
                _row_spec((N, 2)),
                _full_spec((1, H)),
                _full_spec((H, H)),
                _full_spec((1, H)),
                _full_spec((H, 8)),
                _full_spec((1, 8)),
                _full_spec((6, H)),
            ],
            out_specs=[_row_spec((N, 4)), _row_spec((N, 1)),
                       _row_spec((N, H))],
            out_shape=[
                jax.ShapeDtypeStruct((N, 4), jnp.float32),
                jax.ShapeDtypeStruct((N, 1), jnp.float32),
                jax.ShapeDtypeStruct((N, H), jnp.float32),
            ],
        )(rawp, hw2s, z, ld, dinv, yb, b2, M1, mb1, M2, mb2, W1n)

    return call


_coupling_even = _make_coupling(True)
_coupling_odd = _make_coupling(False)


def kernel(x, edge_index, y, batch, W1, b1, W2, b2, M1, mb1, M2, mb2):
    L = W1.shape[0]
    pad = jnp.arange(EP - EDGES, dtype=jnp.int32)
    src1 = jnp.concatenate([edge_index[0], pad % N])
    dst1 = jnp.concatenate([edge_index[1], N + pad % (NP - N)])
    dst2 = dst1.reshape(EP // K, K)
    zeros1 = jnp.zeros((NP,), jnp.float32)
    zerosH = jnp.zeros((NP, H), jnp.float32)
    batch2 = batch.reshape(N, 1)

    degp = _make_deg_kernel()(dst2, zeros1)
    _spmmH = _make_spmm(H)
    dinv, yb, hw1s = _pre(degp, batch2, y, x, W1[0])

    b1r = b1.reshape(L, 1, H)
    b2r = b2.reshape(L, 1, H)
    mb1r = mb1.reshape(L, 1, H)
    mb2r = mb2.reshape(L, 1, 8)

    z = x
    ld = jnp.zeros((N, 1), jnp.float32)
    for i in range(L):
        raw1 = _spmmH(hw1s, src1, dst1, zerosH)
        hw2s = _mid(raw1, hw1s, dinv, b1r[i], W2[i])
        raw2 = _spmmH(hw2s, src1, dst1, zerosH)
        coup = _coupling_even if i % 2 == 0 else _coupling_odd
        z, ld, hw1s = coup(raw2, hw2s, z, ld, dinv, yb, b2r[i],
                           M1[i], mb1r[i], M2[i], mb2r[i], W1[(i + 1) % L])
    return z, ld.reshape(N)
